# baseline (speedup 1.0000x reference)
import functools

import jax
import jax.numpy as jnp
from jax import lax
from jax.experimental import pallas as pl
from jax.experimental.pallas import tpu as pltpu
from jax.experimental.pallas import tpu_sc as plsc

B, T, H = 1024, 200, 768
POS_OFF = 2
N = B * T
NC, NS = 2, 16
NW = NC * NS
ROWS_W = N // NW
SEQ_W = ROWS_W // T
TCH = 40
NTC = T // TCH
NJ = NTC * SEQ_W
VPR = H // 16
HALF = 24
NB = 3

_mesh = plsc.VectorSubcoreMesh(core_axis_name="c", subcore_axis_name="s")


@functools.partial(
    pl.kernel,
    out_type=jax.ShapeDtypeStruct((N, H), jnp.float32),
    mesh=_mesh,
    scratch_types=[
        pltpu.VMEM((ROWS_W,), jnp.int32),
        pltpu.VMEM((TCH, H), jnp.float32),
        pltpu.VMEM((NB, TCH, H), jnp.float32),
        pltpu.SemaphoreType.DMA((NB,)),
        pltpu.SemaphoreType.DMA((NB,)),
    ],
)
def _emb(x_hbm, wtab_hbm, pos_hbm, out_hbm, idx_v, pos_v, ring_v, sg, so):
    wid = lax.axis_index("s") * NC + lax.axis_index("c")
    base = wid * ROWS_W
    pltpu.sync_copy(x_hbm.at[pl.ds(base, ROWS_W)], idx_v)

    def job_loc(j):
        tc = j // SEQ_W
        bl = j - tc * SEQ_W
        return bl * T + tc * TCH

    def gather_desc(j, b):
        return pltpu.make_async_copy(
            wtab_hbm.at[idx_v.at[pl.ds(job_loc(j), TCH)]],
            ring_v.at[b], sg.at[b])

    def out_desc(j, b):
        return pltpu.make_async_copy(
            ring_v.at[b], out_hbm.at[pl.ds(base + job_loc(j), TCH)],
            so.at[b])

    for b in range(NB):
        gather_desc(b, b).start()

    REM = NJ % NB
    MAIN = NJ - REM

    def slot(j, b, refill):
        @pl.when(j % SEQ_W == 0)
        def _pos():
            pltpu.sync_copy(pos_hbm.at[pl.ds((j // SEQ_W) * TCH, TCH)], pos_v)

        gather_desc(j, b).wait()

        @plsc.parallel_loop(0, HALF)
        def _row_a(r):
            for c in range(VPR):
                sl = pl.ds(c * 16, 16)
                plsc.addupdate(ring_v.at[b, r, sl], pos_v[r, sl])

        loc = base + job_loc(j)
        pltpu.make_async_copy(
            ring_v.at[b, pl.ds(0, HALF)],
            out_hbm.at[pl.ds(loc, HALF)], so.at[b]).start()

        @plsc.parallel_loop(HALF, TCH)
        def _row_b(r):
            for c in range(VPR):
                sl = pl.ds(c * 16, 16)
                plsc.addupdate(ring_v.at[b, r, sl], pos_v[r, sl])

        pltpu.make_async_copy(
            ring_v.at[b, pl.ds(HALF, TCH - HALF)],
            out_hbm.at[pl.ds(loc + HALF, TCH - HALF)], so.at[b]).start()
        if refill:
            pb = (b - 1) % NB
            pg = j + NB - 1

            @pl.when(jnp.logical_and(j >= 1, pg < NJ))
            def _refill():
                out_desc(j - 1, pb).wait()
                gather_desc(pg, pb).start()

    @pl.loop(0, MAIN, step=NB)
    def _grp(j0):
        for b in range(NB):
            slot(j0 + b, b, refill=True)

    for k in range(REM):
        j = MAIN + k
        slot(j, j % NB, refill=True)

    for j in range(NJ - NB, NJ):
        out_desc(j, j % NB).wait()


def kernel(x, word_embeddings, position_embeddings):
    xf = x.reshape(N)
    pos2 = lax.slice_in_dim(position_embeddings, POS_OFF, POS_OFF + T, axis=0)
    out = _emb(xf, word_embeddings, pos2)
    return out.reshape(B, T, H)


# speedup vs baseline: 1.0359x; 1.0359x over previous
import functools

import jax
import jax.numpy as jnp
from jax import lax
from jax.experimental import pallas as pl
from jax.experimental.pallas import tpu as pltpu
from jax.experimental.pallas import tpu_sc as plsc

B, T, H = 1024, 200, 768
POS_OFF = 2
N = B * T
NC, NS = 2, 16
NW = NC * NS
ROWS_W = N // NW
SEQ_W = ROWS_W // T
TCH = 40
NTC = T // TCH
NJ = NTC * SEQ_W
VPR = H // 16
NB = 3

_mesh = plsc.VectorSubcoreMesh(core_axis_name="c", subcore_axis_name="s")


@functools.partial(
    pl.kernel,
    out_type=jax.ShapeDtypeStruct((N, H), jnp.float32),
    mesh=_mesh,
    scratch_types=[
        pltpu.VMEM((ROWS_W,), jnp.int32),
        pltpu.VMEM((TCH, H), jnp.float32),
        pltpu.VMEM((NB, TCH, H), jnp.float32),
        pltpu.SemaphoreType.DMA((NB,)),
        pltpu.SemaphoreType.DMA((NB,)),
        pltpu.SemaphoreType.DMA,
    ],
)
def _emb(x_hbm, wtab_hbm, pos_hbm, out_hbm, idx_v, pos_v, ring_v, sg, so,
         sp):
    wid = lax.axis_index("s") * NC + lax.axis_index("c")
    base = wid * ROWS_W
    pltpu.sync_copy(x_hbm.at[pl.ds(base, ROWS_W)], idx_v)

    def job_loc(j):
        tc = j // SEQ_W
        bl = j - tc * SEQ_W
        return bl * T + tc * TCH

    def gather_desc(j, b):
        return pltpu.make_async_copy(
            wtab_hbm.at[idx_v.at[pl.ds(job_loc(j), TCH)]],
            ring_v.at[b], sg.at[b])

    def out_desc(j, b):
        return pltpu.make_async_copy(
            ring_v.at[b], out_hbm.at[pl.ds(base + job_loc(j), TCH)],
            so.at[b])

    for b in range(NB):
        gather_desc(b, b).start()

    REM = NJ % NB
    MAIN = NJ - REM

    def slot(j, b, refill):
        def pos_desc(j):
            return pltpu.make_async_copy(
                pos_hbm.at[pl.ds((j // SEQ_W) * TCH, TCH)], pos_v, sp)

        @pl.when(j % SEQ_W == 0)
        def _pos_start():
            pos_desc(j).start()

        gather_desc(j, b).wait()

        @pl.when(j % SEQ_W == 0)
        def _pos_wait():
            pos_desc(j).wait()

        @plsc.parallel_loop(0, TCH)
        def _row(r):
            for c in range(VPR):
                sl = pl.ds(c * 16, 16)
                plsc.addupdate(ring_v.at[b, r, sl], pos_v[r, sl])

        out_desc(j, b).start()
        if refill:
            pb = (b - 1) % NB
            pg = j + NB - 1

            @pl.when(jnp.logical_and(j >= 1, pg < NJ))
            def _refill():
                out_desc(j - 1, pb).wait()
                gather_desc(pg, pb).start()

    @pl.loop(0, MAIN, step=NB)
    def _grp(j0):
        for b in range(NB):
            slot(j0 + b, b, refill=True)

    for k in range(REM):
        j = MAIN + k
        slot(j, j % NB, refill=True)

    for j in range(NJ - NB, NJ):
        out_desc(j, j % NB).wait()


def kernel(x, word_embeddings, position_embeddings):
    xf = x.reshape(N)
    pos2 = lax.slice_in_dim(position_embeddings, POS_OFF, POS_OFF + T, axis=0)
    out = _emb(xf, word_embeddings, pos2)
    return out.reshape(B, T, H)
